# baseline (speedup 1.0000x reference)
import functools

import jax
import jax.numpy as jnp
from jax import lax
from jax.experimental import pallas as pl
from jax.experimental.pallas import tpu as pltpu
from jax.experimental.pallas import tpu_sc as plsc

N = 1000000
EMBED_DIM = 32
BATCH = 16384
FIELDS = 26

NC = 2
NS = 16
NW = NC * NS

TOTAL = BATCH * FIELDS
PER_W = TOTAL // NW
CHUNK = 128
NCHUNK = PER_W // CHUNK
G = 4
NGROUP = NCHUNK // G
GROWS = G * CHUNK
BBLK = BATCH // CHUNK
OUTROWS = FIELDS * 4 * BBLK

assert PER_W * NW == TOTAL
assert CHUNK * NCHUNK == PER_W
assert G * NGROUP == NCHUNK

VBLK = 32768
NB = 31
VPAD = NB * VBLK


def _tr_body(tt_ref, o_ref):
    for g in range(VBLK // 512):
        m = jnp.concatenate(
            [tt_ref[:, 512 * g + 128 * j : 512 * g + 128 * (j + 1)] for j in range(4)],
            axis=0,
        )
        o_ref[128 * g : 128 * (g + 1), :] = m.T


_transpose = pl.pallas_call(
    _tr_body,
    grid=(NB,),
    in_specs=[pl.BlockSpec((32, VBLK), lambda i: (0, i))],
    out_specs=pl.BlockSpec((VBLK // 4, 128), lambda i: (i, 0)),
    out_shape=jax.ShapeDtypeStruct((NB * VBLK // 4, 128), jnp.float32),
)


def _body(x_hbm, table_hbm, out_hbm, idx_v, rows0, rows1, tb0, tb1,
          gs0, gs1, ws0, ws1):
    c = lax.axis_index("c")
    s = lax.axis_index("s")
    wid = s * NC + c
    pltpu.sync_copy(x_hbm.at[wid], idx_v)

    qbase = wid * NCHUNK

    def fire_gather(g, rows, gsem):
        for b in range(G):
            pltpu.async_copy(
                table_hbm.at[idx_v.at[g * G + b]],
                rows.at[pl.ds(b * CHUNK, CHUNK)],
                gsem,
            )

    def gdrain(rows, gsem):
        for b in range(G):
            pltpu.make_async_copy(
                table_hbm.at[pl.ds(0, CHUNK)],
                rows.at[pl.ds(b * CHUNK, CHUNK)],
                gsem,
            ).wait()

    def wdrain(tb, wsem):
        for b in range(G):
            for k in range(4):
                pltpu.make_async_copy(
                    tb.at[b, pl.ds(8 * k, 8), :], out_hbm.at[0], wsem
                ).wait()

    def half(g, rows, tb, gsem, wsem):
        gdrain(rows, gsem)

        @pl.when(g >= 2)
        def _():
            wdrain(tb, wsem)

        for b in range(G):
            @pl.loop(0, EMBED_DIM, unroll=8)
            def _d(d):
                cols = jnp.full((16,), d, jnp.int32)
                for p in range(8):
                    rids = lax.iota(jnp.int32, 16) + (b * CHUNK + 16 * p)
                    v = plsc.load_gather(rows, [rids, cols])
                    tb[b, d, pl.ds(16 * p, 16)] = v
        q = qbase + g * G
        for b in range(G):
            f = (q + b) // BBLK
            bt = (q + b) % BBLK
            for k in range(4):
                pltpu.async_copy(
                    tb.at[b, pl.ds(8 * k, 8), :],
                    out_hbm.at[f * 512 + k * 128 + bt],
                    wsem,
                )

        @pl.when(g + 2 < NGROUP)
        def _():
            fire_gather(g + 2, rows, gsem)

    fire_gather(0, rows0, gs0)
    fire_gather(1, rows1, gs1)

    @pl.loop(0, NGROUP, step=2)
    def _grp(g0):
        half(g0, rows0, tb0, gs0, ws0)
        half(g0 + 1, rows1, tb1, gs1, ws1)

    wdrain(tb0, ws0)
    wdrain(tb1, ws1)


_mesh = plsc.VectorSubcoreMesh(
    core_axis_name="c", subcore_axis_name="s", num_cores=NC, num_subcores=NS
)

_sc_gather = pl.kernel(
    _body,
    out_type=jax.ShapeDtypeStruct((OUTROWS, 8, CHUNK), jnp.float32),
    mesh=_mesh,
    scratch_types=[
        pltpu.VMEM((NCHUNK, CHUNK), jnp.int32),
        pltpu.VMEM((GROWS, EMBED_DIM), jnp.float32),
        pltpu.VMEM((GROWS, EMBED_DIM), jnp.float32),
        pltpu.VMEM((G, EMBED_DIM, CHUNK), jnp.float32),
        pltpu.VMEM((G, EMBED_DIM, CHUNK), jnp.float32),
        pltpu.SemaphoreType.DMA,
        pltpu.SemaphoreType.DMA,
        pltpu.SemaphoreType.DMA,
        pltpu.SemaphoreType.DMA,
    ],
    compiler_params=pltpu.CompilerParams(
        use_tc_tiling_on_sc=False, needs_layout_passes=False
    ),
)


@jax.jit
def kernel(x, table):
    tbl_rm = _transpose(table.T).reshape(VPAD, EMBED_DIM)
    xi = x.T.astype(jnp.int32)
    idx = ((xi & ~511) | ((xi & 127) << 2) | ((xi >> 7) & 3)).reshape(
        NW, NCHUNK, CHUNK
    )
    out2 = _sc_gather(idx, tbl_rm)
    a5 = out2.reshape(FIELDS, 4, BBLK, 8, CHUNK)
    return a5.transpose(2, 4, 0, 1, 3).reshape(BATCH, FIELDS, EMBED_DIM)


# speedup vs baseline: 1.1141x; 1.1141x over previous
import functools

import jax
import jax.numpy as jnp
from jax import lax
from jax.experimental import pallas as pl
from jax.experimental.pallas import tpu as pltpu
from jax.experimental.pallas import tpu_sc as plsc

N = 1000000
EMBED_DIM = 32
BATCH = 16384
FIELDS = 26

NC = 2
NS = 16
NW = NC * NS

TOTAL = BATCH * FIELDS
PER_W = TOTAL // NW
CHUNK = 128
NCHUNK = PER_W // CHUNK
G = 8
NGROUP = NCHUNK // G

assert PER_W * NW == TOTAL
assert CHUNK * NCHUNK == PER_W
assert G * NGROUP == NCHUNK


GROWS = G * CHUNK

VBLK = 65536
NB = 16
VPAD = NB * VBLK


def _tr_body(tt_ref, o_ref):
    for g in range(VBLK // 512):
        m = jnp.concatenate(
            [tt_ref[:, 512 * g + 128 * j : 512 * g + 128 * (j + 1)] for j in range(4)],
            axis=0,
        )
        o_ref[128 * g : 128 * (g + 1), :] = m.T


_transpose = pl.pallas_call(
    _tr_body,
    grid=(NB,),
    in_specs=[pl.BlockSpec((32, VBLK), lambda i: (0, i))],
    out_specs=pl.BlockSpec((VBLK // 4, 128), lambda i: (i, 0)),
    out_shape=jax.ShapeDtypeStruct((NB * VBLK // 4, 128), jnp.float32),
)


def _body(x_hbm, table_hbm, out_hbm, idx_v, rows0, rows1, gs0, gs1, ws0, ws1):
    c = lax.axis_index("c")
    s = lax.axis_index("s")
    wid = s * NC + c
    base = wid * PER_W
    pltpu.sync_copy(x_hbm.at[wid], idx_v)

    bufs = ((rows0, gs0, ws0), (rows1, gs1, ws1))

    def fire_gather(g):
        buf, gsem, _ = bufs[g % 2]
        return [
            pltpu.async_copy(
                table_hbm.at[idx_v.at[g * G + b]],
                buf.at[pl.ds(b * CHUNK, CHUNK)],
                gsem,
            )
            for b in range(G)
        ]

    gpend = {0: fire_gather(0), 1: fire_gather(1)}
    wpend = {}
    for g in range(NGROUP):
        buf, _, wsem = bufs[g % 2]
        for cp in gpend.pop(g):
            cp.wait()
        wpend[g] = pltpu.async_copy(
            buf, out_hbm.at[pl.ds(base + g * GROWS, GROWS)], wsem
        )
        if g + 2 < NGROUP:
            wpend.pop(g).wait()
            gpend[g + 2] = fire_gather(g + 2)
    for cp in wpend.values():
        cp.wait()


_mesh = plsc.VectorSubcoreMesh(
    core_axis_name="c", subcore_axis_name="s", num_cores=NC, num_subcores=NS
)

_sc_gather = pl.kernel(
    _body,
    out_type=jax.ShapeDtypeStruct((TOTAL, EMBED_DIM), jnp.float32),
    mesh=_mesh,
    scratch_types=[
        pltpu.VMEM((NCHUNK, CHUNK), jnp.int32),
        pltpu.VMEM((GROWS, EMBED_DIM), jnp.float32),
        pltpu.VMEM((GROWS, EMBED_DIM), jnp.float32),
        pltpu.SemaphoreType.DMA,
        pltpu.SemaphoreType.DMA,
        pltpu.SemaphoreType.DMA,
        pltpu.SemaphoreType.DMA,
    ],
    compiler_params=pltpu.CompilerParams(use_tc_tiling_on_sc=False),
)


@jax.jit
def kernel(x, table):
    tbl_rm = _transpose(table.T).reshape(VPAD, EMBED_DIM)
    xi = x.astype(jnp.int32)
    idx = ((xi & ~511) | ((xi & 127) << 2) | ((xi >> 7) & 3)).reshape(
        NW, NCHUNK, CHUNK
    )
    out = _sc_gather(idx, tbl_rm)
    return out.reshape(BATCH, FIELDS, EMBED_DIM)
